# Initial kernel scaffold; baseline (speedup 1.0000x reference)
import jax
import jax.numpy as jnp
from jax.experimental import pallas as pl


def kernel(x, edge_index, edge_attr, params):
    raise NotImplementedError("write your pallas kernel here")


# speedup vs baseline: 2.7042x; 2.7042x over previous
import functools

import jax
import jax.numpy as jnp
from jax import lax
from jax.experimental import pallas as pl
from jax.experimental.pallas import tpu as pltpu
from jax.experimental.pallas import tpu_sc as plsc

N = 10000
E = 320000
D = 128
ED = 16
H = 128
M = 6

NC = 2
NS = 16
NW = NC * NS
EPT = E // NW
G = 80
NCHUNK = EPT // G
NPAD = 10240
NH = NPAD // 2
CG = 2000
NCH2 = EPT // CG
ND = NH // 1024
SLAB = 1024 * ED

NB = 1000
EB = 2000

_F32 = jnp.float32


def _gelu(x):
    return 0.5 * x * (1.0 + lax.erf(x * 0.7071067811865476))


def _layernorm(x, g, b):
    mu = jnp.mean(x, axis=-1, keepdims=True)
    var = jnp.mean((x - mu) ** 2, axis=-1, keepdims=True)
    return (x - mu) * lax.rsqrt(var + 1e-5) * g + b


def _full(shape):
    return pl.BlockSpec(shape, lambda i: (0,) * len(shape))


def _mesh():
    return plsc.VectorSubcoreMesh(core_axis_name="c", subcore_axis_name="s")


def _enc_node(x, w0, b0, w1, b1, wi, bi, wj):
    def body(x_r, w0_r, b0_r, w1_r, b1_r, wi_r, bi_r, wj_r, h_r, a_r, p_r):
        t = _gelu(jnp.dot(x_r[...], w0_r[...], preferred_element_type=_F32) + b0_r[...])
        h = jnp.dot(t, w1_r[...], preferred_element_type=_F32) + b1_r[...]
        h_r[...] = h
        a_r[...] = jnp.dot(h, wi_r[...], preferred_element_type=_F32) + bi_r[...]
        p_r[...] = jnp.dot(h, wj_r[...], preferred_element_type=_F32)

    blk = pl.BlockSpec((NB, D), lambda i: (i, 0))
    sd = jax.ShapeDtypeStruct((N, D), _F32)
    return pl.pallas_call(
        body, grid=(N // NB,),
        in_specs=[blk, _full((D, H)), _full((1, H)), _full((H, D)), _full((1, D)),
                  _full((D, H)), _full((1, H)), _full((D, H))],
        out_specs=[blk, blk, blk],
        out_shape=[sd, sd, sd],
    )(x, w0, b0, w1, b1, wi, bi, wj)


def _enc_edge(ea, w0, b0, w1, b1):
    def body(ea_r, w0_r, b0_r, w1_r, b1_r, e_r):
        t = _gelu(jnp.dot(ea_r[...], w0_r[...], preferred_element_type=_F32) + b0_r[...])
        e_r[...] = jnp.dot(t, w1_r[...], preferred_element_type=_F32) + b1_r[...]

    blk = pl.BlockSpec((EB, ED), lambda i: (i, 0))
    return pl.pallas_call(
        body, grid=(E // EB,),
        in_specs=[blk, _full((ED, H)), _full((1, H)), _full((H, ED)), _full((1, ED))],
        out_specs=blk,
        out_shape=jax.ShapeDtypeStruct((E, ED), _F32),
    )(ea, w0, b0, w1, b1)


def _edge_mlp(pre, e, scale, w0e, w1, b1, w2, b2, g, beta):
    def body(pre_r, e_r, w0e_r, w1_r, b1_r, w2_r, b2_r, g_r, beta_r, msg_r):
        pre0 = pre_r[...] + jnp.dot(e_r[...] * scale, w0e_r[...],
                                    preferred_element_type=_F32)
        u = _gelu(pre0)
        v = _gelu(jnp.dot(u, w1_r[...], preferred_element_type=_F32) + b1_r[...])
        msg = jnp.dot(v, w2_r[...], preferred_element_type=_F32) + b2_r[...]
        msg_r[...] = _layernorm(msg, g_r[...], beta_r[...])

    blkp = pl.BlockSpec((EB, D), lambda i: (i, 0))
    blke = pl.BlockSpec((EB, ED), lambda i: (i, 0))
    return pl.pallas_call(
        body, grid=(E // EB,),
        in_specs=[blkp, blke, _full((ED, H)), _full((H, H)), _full((1, H)),
                  _full((H, ED)), _full((1, ED)), _full((1, ED)), _full((1, ED))],
        out_specs=blke,
        out_shape=jax.ShapeDtypeStruct((E, ED), _F32),
    )(pre, e, w0e, w1, b1, w2, b2, g, beta)


def _combine_rcnt(cnts):
    def body(c_r, o_r):
        s = jnp.sum(c_r[...], axis=0)
        o_r[...] = 1.0 / jnp.maximum(s, 1.0)

    return pl.pallas_call(
        body, grid=(NPAD // 1024,),
        in_specs=[pl.BlockSpec((NW, 1024, ED), lambda i: (0, i, 0))],
        out_specs=pl.BlockSpec((1024, ED), lambda i: (i, 0)),
        out_shape=jax.ShapeDtypeStruct((NPAD, ED), _F32),
    )(cnts)


def _node_common(h_r, s_r, rc_r, nh_r, na_r, nb0_r, n1_r, nb1_r, n2_r, nb2_r,
                 g_r, beta_r):
    h = h_r[...]
    aggr = jnp.sum(s_r[...], axis=0) * rc_r[...]
    t = _gelu(jnp.dot(h, nh_r[...], preferred_element_type=_F32)
              + jnp.dot(aggr, na_r[...], preferred_element_type=_F32) + nb0_r[...])
    t = _gelu(jnp.dot(t, n1_r[...], preferred_element_type=_F32) + nb1_r[...])
    t = jnp.dot(t, n2_r[...], preferred_element_type=_F32) + nb2_r[...]
    return h + _layernorm(t, g_r[...], beta_r[...])


_NODE_SPECS = [
    pl.BlockSpec((NB, D), lambda i: (i, 0)),
    pl.BlockSpec((NW, NB, ED), lambda i: (0, i, 0)),
    pl.BlockSpec((NB, ED), lambda i: (i, 0)),
    _full((D, H)), _full((ED, H)), _full((1, H)),
    _full((H, H)), _full((1, H)),
    _full((H, D)), _full((1, D)),
    _full((1, D)), _full((1, D)),
]


def _node_update(h, sums, rcnt, nh, na, nb0, n1, nb1, n2, nb2, g, beta,
                 wi, bi, wj):
    def body(h_r, s_r, rc_r, nh_r, na_r, nb0_r, n1_r, nb1_r, n2_r, nb2_r,
             g_r, beta_r, wi_r, bi_r, wj_r, h2_r, a_r, p_r):
        h2 = _node_common(h_r, s_r, rc_r, nh_r, na_r, nb0_r, n1_r, nb1_r,
                          n2_r, nb2_r, g_r, beta_r)
        h2_r[...] = h2
        a_r[...] = jnp.dot(h2, wi_r[...], preferred_element_type=_F32) + bi_r[...]
        p_r[...] = jnp.dot(h2, wj_r[...], preferred_element_type=_F32)

    blk = pl.BlockSpec((NB, D), lambda i: (i, 0))
    sd = jax.ShapeDtypeStruct((N, D), _F32)
    return pl.pallas_call(
        body, grid=(N // NB,),
        in_specs=_NODE_SPECS + [_full((D, H)), _full((1, H)), _full((D, H))],
        out_specs=[blk, blk, blk],
        out_shape=[sd, sd, sd],
    )(h, sums, rcnt, nh, na, nb0, n1, nb1, n2, nb2, g, beta, wi, bi, wj)


def _node_update_dec(h, sums, rcnt, nh, na, nb0, n1, nb1, n2, nb2, g, beta,
                     d0, db0, d1, db1):
    def body(h_r, s_r, rc_r, nh_r, na_r, nb0_r, n1_r, nb1_r, n2_r, nb2_r,
             g_r, beta_r, d0_r, db0_r, d1_r, db1_r, o_r):
        h2 = _node_common(h_r, s_r, rc_r, nh_r, na_r, nb0_r, n1_r, nb1_r,
                          n2_r, nb2_r, g_r, beta_r)
        t = _gelu(jnp.dot(h2, d0_r[...], preferred_element_type=_F32) + db0_r[...])
        o_r[...] = jnp.dot(t, d1_r[...], preferred_element_type=_F32) + db1_r[...]

    blk = pl.BlockSpec((NB, D), lambda i: (i, 0))
    return pl.pallas_call(
        body, grid=(N // NB,),
        in_specs=_NODE_SPECS + [_full((D, H)), _full((1, H)),
                                _full((H, D)), _full((1, D))],
        out_specs=blk,
        out_shape=jax.ShapeDtypeStruct((N, D), _F32),
    )(h, sums, rcnt, nh, na, nb0, n1, nb1, n2, nb2, g, beta, d0, db0, d1, db1)


def _sc_gather_add(a, b, dstc, srcc):
    @functools.partial(
        pl.kernel,
        out_type=jax.ShapeDtypeStruct((E, D), _F32),
        mesh=_mesh(),
        scratch_types=[
            pltpu.VMEM((NCHUNK, G), jnp.int32),
            pltpu.VMEM((NCHUNK, G), jnp.int32),
            pltpu.VMEM((G, D), _F32),
            pltpu.VMEM((G, D), _F32),
            pltpu.SemaphoreType.DMA,
            pltpu.SemaphoreType.DMA,
        ])
    def k(a_hbm, b_hbm, dstc_hbm, srcc_hbm, pre_hbm, idxd, idxs, bufa, bufb,
          sema, semb):
        wid = lax.axis_index("s") * NC + lax.axis_index("c")
        base = wid * EPT
        pltpu.sync_copy(dstc_hbm.at[wid], idxd)
        pltpu.sync_copy(srcc_hbm.at[wid], idxs)

        def chunk(c, carry):
            ca = pltpu.async_copy(a_hbm.at[idxd.at[c]], bufa, sema)
            cb = pltpu.async_copy(b_hbm.at[idxs.at[c]], bufb, semb)
            ca.wait()
            cb.wait()

            def row(i, carry2):
                for j in range(D // 16):
                    sl = pl.ds(j * 16, 16)
                    bufa[i, sl] = bufa[i, sl] + bufb[i, sl]
                return carry2

            lax.fori_loop(0, G, row, 0)
            pltpu.sync_copy(bufa, pre_hbm.at[pl.ds(base + c * G, G)])
            return carry

        lax.fori_loop(0, NCHUNK, chunk, 0)

    return k(a, b, dstc, srcc)


_SCAT_PARAMS = pltpu.CompilerParams(needs_layout_passes=False)


def _sc_scatter(msgf, dst):
    @functools.partial(
        pl.kernel,
        out_type=jax.ShapeDtypeStruct((NW, 2 * ND, SLAB), _F32),
        mesh=_mesh(),
        compiler_params=_SCAT_PARAMS,
        scratch_types=[
            pltpu.VMEM(((NH + 8) * ED,), _F32),
            pltpu.VMEM((CG * ED,), _F32),
            pltpu.VMEM((EPT,), jnp.int32),
        ])
    def k(msg_hbm, idx_hbm, out_hbm, acc, mbuf, idxv):
        wid = lax.axis_index("s") * NC + lax.axis_index("c")
        base = wid * EPT
        iot = lax.iota(jnp.int32, 16)
        pltpu.sync_copy(idx_hbm.at[pl.ds(base, EPT)], idxv)
        for p in range(2):
            lo = p * NH

            def zrow(i, carry):
                acc[pl.ds(i * 16, 16)] = jnp.zeros((16,), _F32)
                return carry

            lax.fori_loop(0, (NH + 8) * ED // 16, zrow, 0)

            def chunk(c, carry):
                pltpu.sync_copy(msg_hbm.at[pl.ds((base + c * CG) * ED, CG * ED)],
                                mbuf)

                def grp(g, carry2):
                    dstv = idxv[pl.ds(c * CG + g * 16, 16)]
                    rowv = dstv - lo
                    inr = (rowv >= 0) & (rowv < NH)
                    rowc = jnp.where(inr, rowv, NH)
                    for l in range(16):
                        iv = jnp.full((16,), rowc[l] * ED, jnp.int32) + iot
                        vals = mbuf[pl.ds((g * 16 + l) * ED, 16)]
                        plsc.addupdate_scatter(acc, [iv], vals)
                    return carry2

                lax.fori_loop(0, CG // 16, grp, 0)
                return carry

            lax.fori_loop(0, NCH2, chunk, 0)
            for cc in range(ND):
                pltpu.sync_copy(acc.at[pl.ds(cc * SLAB, SLAB)],
                                out_hbm.at[wid, p * ND + cc])

    return k(msgf, dst).reshape(NW, NPAD, ED)


def _sc_count(dst):
    @functools.partial(
        pl.kernel,
        out_type=jax.ShapeDtypeStruct((NW, 2 * ND, SLAB), _F32),
        mesh=_mesh(),
        compiler_params=_SCAT_PARAMS,
        scratch_types=[
            pltpu.VMEM(((NH + 8) * ED,), _F32),
            pltpu.VMEM((EPT,), jnp.int32),
        ])
    def k(idx_hbm, out_hbm, acc, idxv):
        wid = lax.axis_index("s") * NC + lax.axis_index("c")
        base = wid * EPT
        iot = lax.iota(jnp.int32, 16)
        ones = jnp.ones((16,), _F32)
        pltpu.sync_copy(idx_hbm.at[pl.ds(base, EPT)], idxv)
        for p in range(2):
            lo = p * NH

            def zrow(i, carry):
                acc[pl.ds(i * 16, 16)] = jnp.zeros((16,), _F32)
                return carry

            lax.fori_loop(0, (NH + 8) * ED // 16, zrow, 0)

            def grp(g, carry2):
                dstv = idxv[pl.ds(g * 16, 16)]
                rowv = dstv - lo
                inr = (rowv >= 0) & (rowv < NH)
                rowc = jnp.where(inr, rowv, NH)
                for l in range(16):
                    iv = jnp.full((16,), rowc[l] * ED, jnp.int32) + iot
                    plsc.addupdate_scatter(acc, [iv], ones)
                return carry2

            lax.fori_loop(0, EPT // 16, grp, 0)
            for cc in range(ND):
                pltpu.sync_copy(acc.at[pl.ds(cc * SLAB, SLAB)],
                                out_hbm.at[wid, p * ND + cc])

    return k(dst).reshape(NW, NPAD, ED)


def _r2(v):
    return v.reshape(1, -1)


def kernel(x, edge_index, edge_attr, params):
    src = edge_index[0]
    dst = edge_index[1]
    dstc = dst.reshape(NW, NCHUNK, G)
    srcc = src.reshape(NW, NCHUNK, G)

    lay = params["layers"]
    e0w = lay[0]["e0"]["w"]
    h, a, b = _enc_node(
        x,
        params["node_enc"][0]["w"], _r2(params["node_enc"][0]["b"]),
        params["node_enc"][1]["w"], _r2(params["node_enc"][1]["b"]),
        e0w[:D], _r2(lay[0]["e0"]["b"]), e0w[D:2 * D])
    e = _enc_edge(
        edge_attr,
        params["edge_enc"][0]["w"], _r2(params["edge_enc"][0]["b"]),
        params["edge_enc"][1]["w"], _r2(params["edge_enc"][1]["b"]))
    rcnt = _combine_rcnt(_sc_count(dst))

    for m in range(M):
        lp = lay[m]
        pre = _sc_gather_add(a, b, dstc, srcc)
        msg = _edge_mlp(
            pre, e, 2.0 ** m,
            lp["e0"]["w"][2 * D:], lp["e1"]["w"], _r2(lp["e1"]["b"]),
            lp["e2"]["w"], _r2(lp["e2"]["b"]),
            _r2(lp["eln"]["g"]), _r2(lp["eln"]["b"]))
        sums = _sc_scatter(msg.reshape(-1), dst)
        n0w = lp["n0"]["w"]
        common = (h, sums, rcnt, n0w[:D], n0w[D:], _r2(lp["n0"]["b"]),
                  lp["n1"]["w"], _r2(lp["n1"]["b"]),
                  lp["n2"]["w"], _r2(lp["n2"]["b"]),
                  _r2(lp["nln"]["g"]), _r2(lp["nln"]["b"]))
        if m < M - 1:
            nxt = lay[m + 1]["e0"]
            h, a, b = _node_update(
                *common, nxt["w"][:D], _r2(nxt["b"]), nxt["w"][D:2 * D])
        else:
            out = _node_update_dec(
                *common,
                params["dec"][0]["w"], _r2(params["dec"][0]["b"]),
                params["dec"][1]["w"], _r2(params["dec"][1]["b"]))
    return out


# speedup vs baseline: 3.0568x; 1.1304x over previous
import functools

import jax
import jax.numpy as jnp
from jax import lax
from jax.experimental import pallas as pl
from jax.experimental.pallas import tpu as pltpu
from jax.experimental.pallas import tpu_sc as plsc

N = 10000
E = 320000
D = 128
ED = 16
H = 128
M = 6

NC = 2
NS = 16
NW = NC * NS
EPT = E // NW
G = 80
NCHUNK = EPT // G
NPAD = 10240
NH = NPAD // 2
CG = 2000
NCH2 = EPT // CG
ND = NH // 1024
SLAB = 1024 * ED

NB = 1000
EB = 2000

_F32 = jnp.float32


def _gelu(x):
    return 0.5 * x * (1.0 + lax.erf(x * 0.7071067811865476))


def _layernorm(x, g, b):
    mu = jnp.mean(x, axis=-1, keepdims=True)
    var = jnp.mean((x - mu) ** 2, axis=-1, keepdims=True)
    return (x - mu) * lax.rsqrt(var + 1e-5) * g + b


def _full(shape):
    return pl.BlockSpec(shape, lambda i: (0,) * len(shape))


def _mesh():
    return plsc.VectorSubcoreMesh(core_axis_name="c", subcore_axis_name="s")


def _enc_node(x, w0, b0, w1, b1, wi, bi, wj):
    def body(x_r, w0_r, b0_r, w1_r, b1_r, wi_r, bi_r, wj_r, h_r, a_r, p_r):
        t = _gelu(jnp.dot(x_r[...], w0_r[...], preferred_element_type=_F32) + b0_r[...])
        h = jnp.dot(t, w1_r[...], preferred_element_type=_F32) + b1_r[...]
        h_r[...] = h
        a_r[...] = jnp.dot(h, wi_r[...], preferred_element_type=_F32) + bi_r[...]
        p_r[...] = jnp.dot(h, wj_r[...], preferred_element_type=_F32)

    blk = pl.BlockSpec((NB, D), lambda i: (i, 0))
    sd = jax.ShapeDtypeStruct((N, D), _F32)
    return pl.pallas_call(
        body, grid=(N // NB,),
        in_specs=[blk, _full((D, H)), _full((1, H)), _full((H, D)), _full((1, D)),
                  _full((D, H)), _full((1, H)), _full((D, H))],
        out_specs=[blk, blk, blk],
        out_shape=[sd, sd, sd],
    )(x, w0, b0, w1, b1, wi, bi, wj)


def _enc_edge(ea, w0, b0, w1, b1):
    def body(ea_r, w0_r, b0_r, w1_r, b1_r, e_r):
        t = _gelu(jnp.dot(ea_r[...], w0_r[...], preferred_element_type=_F32) + b0_r[...])
        e_r[...] = jnp.dot(t, w1_r[...], preferred_element_type=_F32) + b1_r[...]

    blk = pl.BlockSpec((EB, ED), lambda i: (i, 0))
    return pl.pallas_call(
        body, grid=(E // EB,),
        in_specs=[blk, _full((ED, H)), _full((1, H)), _full((H, ED)), _full((1, ED))],
        out_specs=blk,
        out_shape=jax.ShapeDtypeStruct((E, ED), _F32),
    )(ea, w0, b0, w1, b1)


def _edge_mlp(pre, e, scale, w0e, w1, b1, w2, b2, g, beta):
    def body(pre_r, e_r, w0e_r, w1_r, b1_r, w2_r, b2_r, g_r, beta_r, msg_r):
        pre0 = pre_r[...] + jnp.dot(e_r[...] * scale, w0e_r[...],
                                    preferred_element_type=_F32)
        u = _gelu(pre0)
        v = _gelu(jnp.dot(u, w1_r[...], preferred_element_type=_F32) + b1_r[...])
        msg = jnp.dot(v, w2_r[...], preferred_element_type=_F32) + b2_r[...]
        msg_r[...] = _layernorm(msg, g_r[...], beta_r[...])

    blkp = pl.BlockSpec((EB, D), lambda i: (i, 0))
    blke = pl.BlockSpec((EB, ED), lambda i: (i, 0))
    return pl.pallas_call(
        body, grid=(E // EB,),
        in_specs=[blkp, blke, _full((ED, H)), _full((H, H)), _full((1, H)),
                  _full((H, ED)), _full((1, ED)), _full((1, ED)), _full((1, ED))],
        out_specs=blke,
        out_shape=jax.ShapeDtypeStruct((E, ED), _F32),
    )(pre, e, w0e, w1, b1, w2, b2, g, beta)


def _combine_rcnt(cnts):
    def body(c_r, o_r):
        s = jnp.sum(c_r[...], axis=0)
        o_r[...] = 1.0 / jnp.maximum(s, 1.0)

    return pl.pallas_call(
        body, grid=(NPAD // 1024,),
        in_specs=[pl.BlockSpec((NW, 1024, ED), lambda i: (0, i, 0))],
        out_specs=pl.BlockSpec((1024, ED), lambda i: (i, 0)),
        out_shape=jax.ShapeDtypeStruct((NPAD, ED), _F32),
    )(cnts)


def _node_common(h_r, s_r, rc_r, nh_r, na_r, nb0_r, n1_r, nb1_r, n2_r, nb2_r,
                 g_r, beta_r):
    h = h_r[...]
    aggr = jnp.sum(s_r[...], axis=0) * rc_r[...]
    t = _gelu(jnp.dot(h, nh_r[...], preferred_element_type=_F32)
              + jnp.dot(aggr, na_r[...], preferred_element_type=_F32) + nb0_r[...])
    t = _gelu(jnp.dot(t, n1_r[...], preferred_element_type=_F32) + nb1_r[...])
    t = jnp.dot(t, n2_r[...], preferred_element_type=_F32) + nb2_r[...]
    return h + _layernorm(t, g_r[...], beta_r[...])


_NODE_SPECS = [
    pl.BlockSpec((NB, D), lambda i: (i, 0)),
    pl.BlockSpec((NW, NB, ED), lambda i: (0, i, 0)),
    pl.BlockSpec((NB, ED), lambda i: (i, 0)),
    _full((D, H)), _full((ED, H)), _full((1, H)),
    _full((H, H)), _full((1, H)),
    _full((H, D)), _full((1, D)),
    _full((1, D)), _full((1, D)),
]


def _node_update(h, sums, rcnt, nh, na, nb0, n1, nb1, n2, nb2, g, beta,
                 wi, bi, wj):
    def body(h_r, s_r, rc_r, nh_r, na_r, nb0_r, n1_r, nb1_r, n2_r, nb2_r,
             g_r, beta_r, wi_r, bi_r, wj_r, h2_r, a_r, p_r):
        h2 = _node_common(h_r, s_r, rc_r, nh_r, na_r, nb0_r, n1_r, nb1_r,
                          n2_r, nb2_r, g_r, beta_r)
        h2_r[...] = h2
        a_r[...] = jnp.dot(h2, wi_r[...], preferred_element_type=_F32) + bi_r[...]
        p_r[...] = jnp.dot(h2, wj_r[...], preferred_element_type=_F32)

    blk = pl.BlockSpec((NB, D), lambda i: (i, 0))
    sd = jax.ShapeDtypeStruct((N, D), _F32)
    return pl.pallas_call(
        body, grid=(N // NB,),
        in_specs=_NODE_SPECS + [_full((D, H)), _full((1, H)), _full((D, H))],
        out_specs=[blk, blk, blk],
        out_shape=[sd, sd, sd],
    )(h, sums, rcnt, nh, na, nb0, n1, nb1, n2, nb2, g, beta, wi, bi, wj)


def _node_update_dec(h, sums, rcnt, nh, na, nb0, n1, nb1, n2, nb2, g, beta,
                     d0, db0, d1, db1):
    def body(h_r, s_r, rc_r, nh_r, na_r, nb0_r, n1_r, nb1_r, n2_r, nb2_r,
             g_r, beta_r, d0_r, db0_r, d1_r, db1_r, o_r):
        h2 = _node_common(h_r, s_r, rc_r, nh_r, na_r, nb0_r, n1_r, nb1_r,
                          n2_r, nb2_r, g_r, beta_r)
        t = _gelu(jnp.dot(h2, d0_r[...], preferred_element_type=_F32) + db0_r[...])
        o_r[...] = jnp.dot(t, d1_r[...], preferred_element_type=_F32) + db1_r[...]

    blk = pl.BlockSpec((NB, D), lambda i: (i, 0))
    return pl.pallas_call(
        body, grid=(N // NB,),
        in_specs=_NODE_SPECS + [_full((D, H)), _full((1, H)),
                                _full((H, D)), _full((1, D))],
        out_specs=blk,
        out_shape=jax.ShapeDtypeStruct((N, D), _F32),
    )(h, sums, rcnt, nh, na, nb0, n1, nb1, n2, nb2, g, beta, d0, db0, d1, db1)


NRING = 4


def _sc_gather_add(a, b, dstc, srcc):
    bufs = []
    for _ in range(NRING):
        bufs += [pltpu.VMEM((G, D), _F32), pltpu.VMEM((G, D), _F32),
                 pltpu.SemaphoreType.DMA, pltpu.SemaphoreType.DMA,
                 pltpu.SemaphoreType.DMA]

    @functools.partial(
        pl.kernel,
        out_type=jax.ShapeDtypeStruct((E, D), _F32),
        mesh=_mesh(),
        scratch_types=[
            pltpu.VMEM((NCHUNK, G), jnp.int32),
            pltpu.VMEM((NCHUNK, G), jnp.int32),
        ] + bufs)
    def k(a_hbm, b_hbm, dstc_hbm, srcc_hbm, pre_hbm, idxd, idxs, *ring):
        wid = lax.axis_index("s") * NC + lax.axis_index("c")
        base = wid * EPT
        pltpu.sync_copy(dstc_hbm.at[wid], idxd)
        pltpu.sync_copy(srcc_hbm.at[wid], idxs)

        def rbuf(r):
            return ring[5 * r], ring[5 * r + 1], ring[5 * r + 2], \
                ring[5 * r + 3], ring[5 * r + 4]

        def fire(r, c):
            ba, bb, sa, sb, _ = rbuf(r)
            pltpu.async_copy(a_hbm.at[idxd.at[c]], ba, sa)
            pltpu.async_copy(b_hbm.at[idxs.at[c]], bb, sb)

        def drain(r, c, first):
            ba, bb, sa, sb, ss = rbuf(r)
            pltpu.make_async_copy(a_hbm.at[idxd.at[c]], ba, sa).wait()
            pltpu.make_async_copy(b_hbm.at[idxs.at[c]], bb, sb).wait()

            def row(i, carry2):
                for j in range(D // 16):
                    sl = pl.ds(j * 16, 16)
                    ba[i, sl] = ba[i, sl] + bb[i, sl]
                return carry2

            lax.fori_loop(0, G, row, 0)

            @pl.when(jnp.logical_not(first))
            def _():
                pltpu.make_async_copy(ba, pre_hbm.at[pl.ds(0, G)], ss).wait()

            pltpu.async_copy(ba, pre_hbm.at[pl.ds(base + c * G, G)], ss)

        for r in range(NRING):
            fire(r, r)

        NFULL = (NCHUNK // NRING) * NRING

        def body4(cc, carry):
            for r in range(NRING):
                c = cc * NRING + r
                drain(r, c, cc == 0)

                @pl.when(c + NRING < NCHUNK)
                def _():
                    fire(r, c + NRING)
            return carry

        lax.fori_loop(0, NFULL // NRING, body4, 0)
        for c in range(NFULL, NCHUNK):
            drain(c - NFULL, c, False)
        for r in range(NRING):
            ba, _, _, _, ss = rbuf(r)
            pltpu.make_async_copy(ba, pre_hbm.at[pl.ds(0, G)], ss).wait()

    return k(a, b, dstc, srcc)


_SCAT_PARAMS = pltpu.CompilerParams(needs_layout_passes=False)


def _sc_scatter(msgf, dst):
    @functools.partial(
        pl.kernel,
        out_type=jax.ShapeDtypeStruct((NW, 2 * ND, SLAB), _F32),
        mesh=_mesh(),
        compiler_params=_SCAT_PARAMS,
        scratch_types=[
            pltpu.VMEM(((NH + 8) * ED,), _F32),
            pltpu.VMEM((CG * ED,), _F32),
            pltpu.VMEM((EPT,), jnp.int32),
        ])
    def k(msg_hbm, idx_hbm, out_hbm, acc, mbuf, idxv):
        wid = lax.axis_index("s") * NC + lax.axis_index("c")
        base = wid * EPT
        iot = lax.iota(jnp.int32, 16)
        pltpu.sync_copy(idx_hbm.at[pl.ds(base, EPT)], idxv)
        for p in range(2):
            lo = p * NH

            def zrow(i, carry):
                acc[pl.ds(i * 16, 16)] = jnp.zeros((16,), _F32)
                return carry

            lax.fori_loop(0, (NH + 8) * ED // 16, zrow, 0)

            def chunk(c, carry):
                pltpu.sync_copy(msg_hbm.at[pl.ds((base + c * CG) * ED, CG * ED)],
                                mbuf)

                def grp(g, carry2):
                    dstv = idxv[pl.ds(c * CG + g * 16, 16)]
                    rowv = dstv - lo
                    inr = (rowv >= 0) & (rowv < NH)
                    rowc = jnp.where(inr, rowv, NH)
                    for l in range(16):
                        iv = jnp.full((16,), rowc[l] * ED, jnp.int32) + iot
                        vals = mbuf[pl.ds((g * 16 + l) * ED, 16)]
                        plsc.addupdate_scatter(acc, [iv], vals)
                    return carry2

                lax.fori_loop(0, CG // 16, grp, 0)
                return carry

            lax.fori_loop(0, NCH2, chunk, 0)
            for cc in range(ND):
                pltpu.sync_copy(acc.at[pl.ds(cc * SLAB, SLAB)],
                                out_hbm.at[wid, p * ND + cc])

    return k(msgf, dst).reshape(NW, NPAD, ED)


def _sc_count(dst):
    @functools.partial(
        pl.kernel,
        out_type=jax.ShapeDtypeStruct((NW, 2 * ND, SLAB), _F32),
        mesh=_mesh(),
        compiler_params=_SCAT_PARAMS,
        scratch_types=[
            pltpu.VMEM(((NH + 8) * ED,), _F32),
            pltpu.VMEM((EPT,), jnp.int32),
        ])
    def k(idx_hbm, out_hbm, acc, idxv):
        wid = lax.axis_index("s") * NC + lax.axis_index("c")
        base = wid * EPT
        iot = lax.iota(jnp.int32, 16)
        ones = jnp.ones((16,), _F32)
        pltpu.sync_copy(idx_hbm.at[pl.ds(base, EPT)], idxv)
        for p in range(2):
            lo = p * NH

            def zrow(i, carry):
                acc[pl.ds(i * 16, 16)] = jnp.zeros((16,), _F32)
                return carry

            lax.fori_loop(0, (NH + 8) * ED // 16, zrow, 0)

            def grp(g, carry2):
                dstv = idxv[pl.ds(g * 16, 16)]
                rowv = dstv - lo
                inr = (rowv >= 0) & (rowv < NH)
                rowc = jnp.where(inr, rowv, NH)
                for l in range(16):
                    iv = jnp.full((16,), rowc[l] * ED, jnp.int32) + iot
                    plsc.addupdate_scatter(acc, [iv], ones)
                return carry2

            lax.fori_loop(0, EPT // 16, grp, 0)
            for cc in range(ND):
                pltpu.sync_copy(acc.at[pl.ds(cc * SLAB, SLAB)],
                                out_hbm.at[wid, p * ND + cc])

    return k(dst).reshape(NW, NPAD, ED)


def _r2(v):
    return v.reshape(1, -1)


def kernel(x, edge_index, edge_attr, params):
    src = edge_index[0]
    dst = edge_index[1]
    dstc = dst.reshape(NW, NCHUNK, G)
    srcc = src.reshape(NW, NCHUNK, G)

    lay = params["layers"]
    e0w = lay[0]["e0"]["w"]
    h, a, b = _enc_node(
        x,
        params["node_enc"][0]["w"], _r2(params["node_enc"][0]["b"]),
        params["node_enc"][1]["w"], _r2(params["node_enc"][1]["b"]),
        e0w[:D], _r2(lay[0]["e0"]["b"]), e0w[D:2 * D])
    e = _enc_edge(
        edge_attr,
        params["edge_enc"][0]["w"], _r2(params["edge_enc"][0]["b"]),
        params["edge_enc"][1]["w"], _r2(params["edge_enc"][1]["b"]))
    rcnt = _combine_rcnt(_sc_count(dst))

    for m in range(M):
        lp = lay[m]
        pre = _sc_gather_add(a, b, dstc, srcc)
        msg = _edge_mlp(
            pre, e, 2.0 ** m,
            lp["e0"]["w"][2 * D:], lp["e1"]["w"], _r2(lp["e1"]["b"]),
            lp["e2"]["w"], _r2(lp["e2"]["b"]),
            _r2(lp["eln"]["g"]), _r2(lp["eln"]["b"]))
        sums = _sc_scatter(msg.reshape(-1), dst)
        n0w = lp["n0"]["w"]
        common = (h, sums, rcnt, n0w[:D], n0w[D:], _r2(lp["n0"]["b"]),
                  lp["n1"]["w"], _r2(lp["n1"]["b"]),
                  lp["n2"]["w"], _r2(lp["n2"]["b"]),
                  _r2(lp["nln"]["g"]), _r2(lp["nln"]["b"]))
        if m < M - 1:
            nxt = lay[m + 1]["e0"]
            h, a, b = _node_update(
                *common, nxt["w"][:D], _r2(nxt["b"]), nxt["w"][D:2 * D])
        else:
            out = _node_update_dec(
                *common,
                params["dec"][0]["w"], _r2(params["dec"][0]["b"]),
                params["dec"][1]["w"], _r2(params["dec"][1]["b"]))
    return out


# speedup vs baseline: 3.0580x; 1.0004x over previous
import functools

import jax
import jax.numpy as jnp
from jax import lax
from jax.experimental import pallas as pl
from jax.experimental.pallas import tpu as pltpu
from jax.experimental.pallas import tpu_sc as plsc

N = 10000
E = 320000
D = 128
ED = 16
H = 128
M = 6

NC = 2
NS = 16
NW = NC * NS
EPT = E // NW
G = 80
NCHUNK = EPT // G
NPAD = 10240
NH = NPAD // 2
CG = 2000
NCH2 = EPT // CG
ND = NH // 1024
SLAB = 1024 * ED

NB = 1000
EB = 2000

_F32 = jnp.float32


def _gelu(x):
    return 0.5 * x * (1.0 + lax.erf(x * 0.7071067811865476))


def _layernorm(x, g, b):
    mu = jnp.mean(x, axis=-1, keepdims=True)
    var = jnp.mean((x - mu) ** 2, axis=-1, keepdims=True)
    return (x - mu) * lax.rsqrt(var + 1e-5) * g + b


def _full(shape):
    return pl.BlockSpec(shape, lambda i: (0,) * len(shape))


def _mesh():
    return plsc.VectorSubcoreMesh(core_axis_name="c", subcore_axis_name="s")


def _enc_node(x, w0, b0, w1, b1, wi, bi, wj):
    def body(x_r, w0_r, b0_r, w1_r, b1_r, wi_r, bi_r, wj_r, h_r, a_r, p_r):
        t = _gelu(jnp.dot(x_r[...], w0_r[...], preferred_element_type=_F32) + b0_r[...])
        h = jnp.dot(t, w1_r[...], preferred_element_type=_F32) + b1_r[...]
        h_r[...] = h
        a_r[...] = jnp.dot(h, wi_r[...], preferred_element_type=_F32) + bi_r[...]
        p_r[...] = jnp.dot(h, wj_r[...], preferred_element_type=_F32)

    blk = pl.BlockSpec((NB, D), lambda i: (i, 0))
    sd = jax.ShapeDtypeStruct((N, D), _F32)
    return pl.pallas_call(
        body, grid=(N // NB,),
        in_specs=[blk, _full((D, H)), _full((1, H)), _full((H, D)), _full((1, D)),
                  _full((D, H)), _full((1, H)), _full((D, H))],
        out_specs=[blk, blk, blk],
        out_shape=[sd, sd, sd],
    )(x, w0, b0, w1, b1, wi, bi, wj)


def _enc_edge(ea, w0, b0, w1, b1):
    def body(ea_r, w0_r, b0_r, w1_r, b1_r, e_r):
        t = _gelu(jnp.dot(ea_r[...], w0_r[...], preferred_element_type=_F32) + b0_r[...])
        e_r[...] = jnp.dot(t, w1_r[...], preferred_element_type=_F32) + b1_r[...]

    blk = pl.BlockSpec((EB, ED), lambda i: (i, 0))
    return pl.pallas_call(
        body, grid=(E // EB,),
        in_specs=[blk, _full((ED, H)), _full((1, H)), _full((H, ED)), _full((1, ED))],
        out_specs=blk,
        out_shape=jax.ShapeDtypeStruct((E, ED), _F32),
    )(ea, w0, b0, w1, b1)


def _edge_mlp(pre, e, scale, w0e, w1, b1, w2, b2, g, beta):
    def body(pre_r, e_r, w0e_r, w1_r, b1_r, w2_r, b2_r, g_r, beta_r, msg_r):
        pre0 = pre_r[...] + jnp.dot(e_r[...] * scale, w0e_r[...],
                                    preferred_element_type=_F32)
        u = _gelu(pre0)
        v = _gelu(jnp.dot(u, w1_r[...], preferred_element_type=_F32) + b1_r[...])
        msg = jnp.dot(v, w2_r[...], preferred_element_type=_F32) + b2_r[...]
        msg_r[...] = _layernorm(msg, g_r[...], beta_r[...])

    blkp = pl.BlockSpec((EB, D), lambda i: (i, 0))
    blke = pl.BlockSpec((EB, ED), lambda i: (i, 0))
    return pl.pallas_call(
        body, grid=(E // EB,),
        in_specs=[blkp, blke, _full((ED, H)), _full((H, H)), _full((1, H)),
                  _full((H, ED)), _full((1, ED)), _full((1, ED)), _full((1, ED))],
        out_specs=blke,
        out_shape=jax.ShapeDtypeStruct((E, ED), _F32),
    )(pre, e, w0e, w1, b1, w2, b2, g, beta)


def _combine_rcnt(cnts):
    def body(c_r, o_r):
        s = jnp.sum(c_r[...], axis=0)
        o_r[...] = 1.0 / jnp.maximum(s, 1.0)

    return pl.pallas_call(
        body, grid=(NPAD // 1024,),
        in_specs=[pl.BlockSpec((NW, 1024, ED), lambda i: (0, i, 0))],
        out_specs=pl.BlockSpec((1024, ED), lambda i: (i, 0)),
        out_shape=jax.ShapeDtypeStruct((NPAD, ED), _F32),
    )(cnts)


def _node_common(h_r, s_r, rc_r, nh_r, na_r, nb0_r, n1_r, nb1_r, n2_r, nb2_r,
                 g_r, beta_r):
    h = h_r[...]
    aggr = jnp.sum(s_r[...], axis=0) * rc_r[...]
    t = _gelu(jnp.dot(h, nh_r[...], preferred_element_type=_F32)
              + jnp.dot(aggr, na_r[...], preferred_element_type=_F32) + nb0_r[...])
    t = _gelu(jnp.dot(t, n1_r[...], preferred_element_type=_F32) + nb1_r[...])
    t = jnp.dot(t, n2_r[...], preferred_element_type=_F32) + nb2_r[...]
    return h + _layernorm(t, g_r[...], beta_r[...])


_NODE_SPECS = [
    pl.BlockSpec((NB, D), lambda i: (i, 0)),
    pl.BlockSpec((NW, NB, ED), lambda i: (0, i, 0)),
    pl.BlockSpec((NB, ED), lambda i: (i, 0)),
    _full((D, H)), _full((ED, H)), _full((1, H)),
    _full((H, H)), _full((1, H)),
    _full((H, D)), _full((1, D)),
    _full((1, D)), _full((1, D)),
]


def _node_update(h, sums, rcnt, nh, na, nb0, n1, nb1, n2, nb2, g, beta,
                 wi, bi, wj):
    def body(h_r, s_r, rc_r, nh_r, na_r, nb0_r, n1_r, nb1_r, n2_r, nb2_r,
             g_r, beta_r, wi_r, bi_r, wj_r, h2_r, a_r, p_r):
        h2 = _node_common(h_r, s_r, rc_r, nh_r, na_r, nb0_r, n1_r, nb1_r,
                          n2_r, nb2_r, g_r, beta_r)
        h2_r[...] = h2
        a_r[...] = jnp.dot(h2, wi_r[...], preferred_element_type=_F32) + bi_r[...]
        p_r[...] = jnp.dot(h2, wj_r[...], preferred_element_type=_F32)

    blk = pl.BlockSpec((NB, D), lambda i: (i, 0))
    sd = jax.ShapeDtypeStruct((N, D), _F32)
    return pl.pallas_call(
        body, grid=(N // NB,),
        in_specs=_NODE_SPECS + [_full((D, H)), _full((1, H)), _full((D, H))],
        out_specs=[blk, blk, blk],
        out_shape=[sd, sd, sd],
    )(h, sums, rcnt, nh, na, nb0, n1, nb1, n2, nb2, g, beta, wi, bi, wj)


def _node_update_dec(h, sums, rcnt, nh, na, nb0, n1, nb1, n2, nb2, g, beta,
                     d0, db0, d1, db1):
    def body(h_r, s_r, rc_r, nh_r, na_r, nb0_r, n1_r, nb1_r, n2_r, nb2_r,
             g_r, beta_r, d0_r, db0_r, d1_r, db1_r, o_r):
        h2 = _node_common(h_r, s_r, rc_r, nh_r, na_r, nb0_r, n1_r, nb1_r,
                          n2_r, nb2_r, g_r, beta_r)
        t = _gelu(jnp.dot(h2, d0_r[...], preferred_element_type=_F32) + db0_r[...])
        o_r[...] = jnp.dot(t, d1_r[...], preferred_element_type=_F32) + db1_r[...]

    blk = pl.BlockSpec((NB, D), lambda i: (i, 0))
    return pl.pallas_call(
        body, grid=(N // NB,),
        in_specs=_NODE_SPECS + [_full((D, H)), _full((1, H)),
                                _full((H, D)), _full((1, D))],
        out_specs=blk,
        out_shape=jax.ShapeDtypeStruct((N, D), _F32),
    )(h, sums, rcnt, nh, na, nb0, n1, nb1, n2, nb2, g, beta, d0, db0, d1, db1)


NRING = 4


def _sc_gather_add(a, b, dstc, srcc):
    bufs = []
    for _ in range(NRING):
        bufs += [pltpu.VMEM((G, D), _F32), pltpu.VMEM((G, D), _F32),
                 pltpu.SemaphoreType.DMA, pltpu.SemaphoreType.DMA,
                 pltpu.SemaphoreType.DMA]

    @functools.partial(
        pl.kernel,
        out_type=jax.ShapeDtypeStruct((E, D), _F32),
        mesh=_mesh(),
        scratch_types=[
            pltpu.VMEM((NCHUNK, G), jnp.int32),
            pltpu.VMEM((NCHUNK, G), jnp.int32),
        ] + bufs)
    def k(a_hbm, b_hbm, dstc_hbm, srcc_hbm, pre_hbm, idxd, idxs, *ring):
        wid = lax.axis_index("s") * NC + lax.axis_index("c")
        base = wid * EPT
        pltpu.sync_copy(dstc_hbm.at[wid], idxd)
        pltpu.sync_copy(srcc_hbm.at[wid], idxs)

        def rbuf(r):
            return ring[5 * r], ring[5 * r + 1], ring[5 * r + 2], \
                ring[5 * r + 3], ring[5 * r + 4]

        def fire(r, c):
            ba, bb, sa, sb, _ = rbuf(r)
            pltpu.async_copy(a_hbm.at[idxd.at[c]], ba, sa)
            pltpu.async_copy(b_hbm.at[idxs.at[c]], bb, sb)

        def drain(r, c, first):
            ba, bb, sa, sb, ss = rbuf(r)
            pltpu.make_async_copy(a_hbm.at[idxd.at[c]], ba, sa).wait()
            pltpu.make_async_copy(b_hbm.at[idxs.at[c]], bb, sb).wait()

            def row(i, carry2):
                for j in range(D // 16):
                    sl = pl.ds(j * 16, 16)
                    ba[i, sl] = ba[i, sl] + bb[i, sl]
                return carry2

            lax.fori_loop(0, G, row, 0)

            @pl.when(jnp.logical_not(first))
            def _():
                pltpu.make_async_copy(ba, pre_hbm.at[pl.ds(0, G)], ss).wait()

            pltpu.async_copy(ba, pre_hbm.at[pl.ds(base + c * G, G)], ss)

        for r in range(NRING):
            fire(r, r)

        NFULL = (NCHUNK // NRING) * NRING

        def body4(cc, carry):
            for r in range(NRING):
                c = cc * NRING + r
                drain(r, c, cc == 0)

                @pl.when(c + NRING < NCHUNK)
                def _():
                    fire(r, c + NRING)
            return carry

        lax.fori_loop(0, NFULL // NRING, body4, 0)
        for c in range(NFULL, NCHUNK):
            drain(c - NFULL, c, False)
        for r in range(NRING):
            ba, _, _, _, ss = rbuf(r)
            pltpu.make_async_copy(ba, pre_hbm.at[pl.ds(0, G)], ss).wait()

    return k(a, b, dstc, srcc)


_SCAT_PARAMS = pltpu.CompilerParams(needs_layout_passes=False)


def _sc_scatter(msgf, dst):
    @functools.partial(
        pl.kernel,
        out_type=jax.ShapeDtypeStruct((NW, 2 * ND, SLAB), _F32),
        mesh=_mesh(),
        compiler_params=_SCAT_PARAMS,
        scratch_types=[
            pltpu.VMEM(((NH + 8) * ED,), _F32),
            pltpu.VMEM((CG * ED,), _F32),
            pltpu.VMEM((EPT,), jnp.int32),
        ])
    def k(msg_hbm, idx_hbm, out_hbm, acc, mbuf, idxv):
        wid = lax.axis_index("s") * NC + lax.axis_index("c")
        base = wid * EPT
        iot = lax.iota(jnp.int32, 16)
        lsel = [jnp.full((16,), l, jnp.int32) for l in range(16)]
        pltpu.sync_copy(idx_hbm.at[pl.ds(base, EPT)], idxv)
        for p in range(2):
            lo = p * NH

            def zrow(i, carry):
                acc[pl.ds(i * 16, 16)] = jnp.zeros((16,), _F32)
                return carry

            lax.fori_loop(0, (NH + 8) * ED // 16, zrow, 0)

            def chunk(c, carry):
                pltpu.sync_copy(msg_hbm.at[pl.ds((base + c * CG) * ED, CG * ED)],
                                mbuf)

                def grp(g, carry2):
                    dstv = idxv[pl.ds(c * CG + g * 16, 16)]
                    rowv = dstv - lo
                    inr = (rowv >= 0) & (rowv < NH)
                    rowed = jnp.where(inr, rowv, NH) * ED
                    for l in range(16):
                        bc = lax.gather(
                            rowed, lsel[l][:, None],
                            lax.GatherDimensionNumbers(
                                offset_dims=(), collapsed_slice_dims=(0,),
                                start_index_map=(0,)),
                            slice_sizes=(1,),
                            mode=lax.GatherScatterMode.PROMISE_IN_BOUNDS)
                        vals = mbuf[pl.ds((g * 16 + l) * ED, 16)]
                        plsc.addupdate_scatter(acc, [bc + iot], vals)
                    return carry2

                lax.fori_loop(0, CG // 16, grp, 0)
                return carry

            lax.fori_loop(0, NCH2, chunk, 0)
            for cc in range(ND):
                pltpu.sync_copy(acc.at[pl.ds(cc * SLAB, SLAB)],
                                out_hbm.at[wid, p * ND + cc])

    return k(msgf, dst).reshape(NW, NPAD, ED)


def _sc_count(dst):
    @functools.partial(
        pl.kernel,
        out_type=jax.ShapeDtypeStruct((NW, 2 * ND, SLAB), _F32),
        mesh=_mesh(),
        compiler_params=_SCAT_PARAMS,
        scratch_types=[
            pltpu.VMEM(((NH + 8) * ED,), _F32),
            pltpu.VMEM((EPT,), jnp.int32),
        ])
    def k(idx_hbm, out_hbm, acc, idxv):
        wid = lax.axis_index("s") * NC + lax.axis_index("c")
        base = wid * EPT
        iot = lax.iota(jnp.int32, 16)
        ones = jnp.ones((16,), _F32)
        pltpu.sync_copy(idx_hbm.at[pl.ds(base, EPT)], idxv)
        for p in range(2):
            lo = p * NH

            def zrow(i, carry):
                acc[pl.ds(i * 16, 16)] = jnp.zeros((16,), _F32)
                return carry

            lax.fori_loop(0, (NH + 8) * ED // 16, zrow, 0)

            def grp(g, carry2):
                dstv = idxv[pl.ds(g * 16, 16)]
                rowv = dstv - lo
                inr = (rowv >= 0) & (rowv < NH)
                rowc = jnp.where(inr, rowv, NH)
                for l in range(16):
                    iv = jnp.full((16,), rowc[l] * ED, jnp.int32) + iot
                    plsc.addupdate_scatter(acc, [iv], ones)
                return carry2

            lax.fori_loop(0, EPT // 16, grp, 0)
            for cc in range(ND):
                pltpu.sync_copy(acc.at[pl.ds(cc * SLAB, SLAB)],
                                out_hbm.at[wid, p * ND + cc])

    return k(dst).reshape(NW, NPAD, ED)


def _r2(v):
    return v.reshape(1, -1)


def kernel(x, edge_index, edge_attr, params):
    src = edge_index[0]
    dst = edge_index[1]
    dstc = dst.reshape(NW, NCHUNK, G)
    srcc = src.reshape(NW, NCHUNK, G)

    lay = params["layers"]
    e0w = lay[0]["e0"]["w"]
    h, a, b = _enc_node(
        x,
        params["node_enc"][0]["w"], _r2(params["node_enc"][0]["b"]),
        params["node_enc"][1]["w"], _r2(params["node_enc"][1]["b"]),
        e0w[:D], _r2(lay[0]["e0"]["b"]), e0w[D:2 * D])
    e = _enc_edge(
        edge_attr,
        params["edge_enc"][0]["w"], _r2(params["edge_enc"][0]["b"]),
        params["edge_enc"][1]["w"], _r2(params["edge_enc"][1]["b"]))
    rcnt = _combine_rcnt(_sc_count(dst))

    for m in range(M):
        lp = lay[m]
        pre = _sc_gather_add(a, b, dstc, srcc)
        msg = _edge_mlp(
            pre, e, 2.0 ** m,
            lp["e0"]["w"][2 * D:], lp["e1"]["w"], _r2(lp["e1"]["b"]),
            lp["e2"]["w"], _r2(lp["e2"]["b"]),
            _r2(lp["eln"]["g"]), _r2(lp["eln"]["b"]))
        sums = _sc_scatter(msg.reshape(-1), dst)
        n0w = lp["n0"]["w"]
        common = (h, sums, rcnt, n0w[:D], n0w[D:], _r2(lp["n0"]["b"]),
                  lp["n1"]["w"], _r2(lp["n1"]["b"]),
                  lp["n2"]["w"], _r2(lp["n2"]["b"]),
                  _r2(lp["nln"]["g"]), _r2(lp["nln"]["b"]))
        if m < M - 1:
            nxt = lay[m + 1]["e0"]
            h, a, b = _node_update(
                *common, nxt["w"][:D], _r2(nxt["b"]), nxt["w"][D:2 * D])
        else:
            out = _node_update_dec(
                *common,
                params["dec"][0]["w"], _r2(params["dec"][0]["b"]),
                params["dec"][1]["w"], _r2(params["dec"][1]["b"]))
    return out
